# Initial kernel scaffold; baseline (speedup 1.0000x reference)
import jax
import jax.numpy as jnp
from jax.experimental import pallas as pl


def kernel(X, F, Q_S, Lambda_S, B_w):
    raise NotImplementedError("write your pallas kernel here")


# speedup vs baseline: 2.7314x; 2.7314x over previous
import functools

import jax
import jax.numpy as jnp
from jax.experimental import pallas as pl
from jax.experimental.pallas import tpu as pltpu

GAMMA = 0.8
EPS = 1e-12
T_SOLVE = 7


BN = 4096


def _pass1_kernel(n, x_ref, qs_ref, f_ref, lam_ref, y_ref, w_acc):
    i = pl.program_id(0)
    nb = pl.num_programs(0)

    @pl.when(i == 0)
    def _init():
        w_acc[...] = jnp.zeros_like(w_acc)

    x = x_ref[...]
    qs = qs_ref[...]
    if n % BN != 0:
        col = i * BN + jax.lax.broadcasted_iota(jnp.int32, x.shape, 1)
        x = jnp.where(col < n, x, 0.0)
        row = i * BN + jax.lax.broadcasted_iota(jnp.int32, qs.shape, 0)
        qs = jnp.where(row < n, qs, 0.0)
    w_acc[...] += jnp.dot(x, qs, preferred_element_type=jnp.float32)

    @pl.when(i == nb - 1)
    def _solve():
        f = f_ref[...]
        ff = jax.lax.dot_general(f, f, (((0,), (0,)), ((), ())),
                                 preferred_element_type=jnp.float32)
        a = ff / (jnp.sqrt(jnp.sum(ff * ff)) + EPS)
        y = w_acc[...]
        p = a
        cp = GAMMA * lam_ref[...]
        for _ in range(T_SOLVE):
            y = y + jnp.dot(p, y, preferred_element_type=jnp.float32) * cp
            p = jnp.dot(p, p, preferred_element_type=jnp.float32)
            cp = cp * cp
        y_ref[...] = y


def _pass2_kernel(qs_ref, y_ref, bw_ref, out_ref):
    zt = jax.lax.dot_general(qs_ref[...], y_ref[...],
                             (((1,), (1,)), ((), ())),
                             preferred_element_type=jnp.float32)
    nrm = jnp.maximum(jnp.sqrt(jnp.sum(zt * zt, axis=1, keepdims=True)), EPS)
    out_ref[...] = jax.lax.dot_general(zt / nrm, bw_ref[...],
                                       (((1,), (1,)), ((), ())),
                                       preferred_element_type=jnp.float32)


def kernel(X, F, Q_S, Lambda_S, B_w):
    m, n = X.shape
    k = Q_S.shape[1]
    m_y = B_w.shape[0]
    bn = BN
    nb = pl.cdiv(n, bn)
    lam = Lambda_S.reshape(1, k)

    y = pl.pallas_call(
        functools.partial(_pass1_kernel, n),
        grid=(nb,),
        in_specs=[
            pl.BlockSpec((m, bn), lambda i: (0, i)),
            pl.BlockSpec((bn, k), lambda i: (i, 0)),
            pl.BlockSpec((m, m), lambda i: (0, 0)),
            pl.BlockSpec((1, k), lambda i: (0, 0)),
        ],
        out_specs=pl.BlockSpec((m, k), lambda i: (0, 0)),
        out_shape=jax.ShapeDtypeStruct((m, k), jnp.float32),
        scratch_shapes=[pltpu.VMEM((m, k), jnp.float32)],
    )(X, Q_S, F, lam)

    out = pl.pallas_call(
        _pass2_kernel,
        grid=(nb,),
        in_specs=[
            pl.BlockSpec((bn, k), lambda i: (i, 0)),
            pl.BlockSpec((m, k), lambda i: (0, 0)),
            pl.BlockSpec((m_y, m), lambda i: (0, 0)),
        ],
        out_specs=pl.BlockSpec((bn, m_y), lambda i: (i, 0)),
        out_shape=jax.ShapeDtypeStruct((n, m_y), jnp.float32),
        compiler_params=pltpu.CompilerParams(
            dimension_semantics=("parallel",)),
    )(Q_S, y, B_w)
    return out
